# baseline (speedup 1.0000x reference)
import dataclasses
import functools
import math

import jax
import jax.numpy as jnp
import numpy as np
from jax import lax
from jax.experimental import pallas as pl
from jax.experimental.pallas import tpu as pltpu
from jax.experimental.pallas import tpu_sc as plsc

N = 10000
E = 160000
D_MODEL = 256
N_HEADS = 8
D_HEAD = D_MODEL // N_HEADS
N_LAYERS = 2
CUTOFF = 8.0
STEP = 0.2
NUM_BASIS = int(CUTOFF / STEP) + 1
SCALE = D_HEAD ** (-0.5)

_SC_SUBCORES = 16
_SC_CORES = 2
_ACC_ROWS = 10240
_ROWS_PER_SUBCORE = _ACC_ROWS // _SC_SUBCORES
_WINDOW = 128
_MSG_W = 128
_EMB_PAD = 10240
_E_PAD = 163840
_STEPS_PER_TILE = _E_PAD // 32 // _WINDOW
_HALF = D_MODEL // 2

_HIGHEST = lax.Precision.HIGHEST

_P = np.array([(i % _HALF // 16) * D_HEAD + (i // _HALF) * (D_HEAD // 2)
               + i % 16 for i in range(D_MODEL)], dtype=np.int32)


def _matmul(a, b):
    return lax.dot_general(a, b, (((1,), (0,)), ((), ())),
                           precision=_HIGHEST,
                           preferred_element_type=jnp.float32)


def _half_head_matrix():
    lane = lax.broadcasted_iota(jnp.int32, (_HALF, N_HEADS), 0)
    head = lax.broadcasted_iota(jnp.int32, (_HALF, N_HEADS), 1)
    return (lane // 16 == head).astype(jnp.float32)


def _head_half_matrix():
    lane = lax.broadcasted_iota(jnp.int32, (N_HEADS, _HALF), 1)
    head = lax.broadcasted_iota(jnp.int32, (N_HEADS, _HALF), 0)
    return (lane // 16 == head).astype(jnp.float32)


def _head_split_matrix():
    lane = lax.broadcasted_iota(jnp.int32, (N_HEADS, D_MODEL), 1)
    head = lax.broadcasted_iota(jnp.int32, (N_HEADS, D_MODEL), 0)
    return (lane % _HALF // 16 == head).astype(jnp.float32)


def _pack_bf16(lo, hi):
    lo16 = lax.bitcast_convert_type(
        lo.astype(jnp.bfloat16).astype(jnp.float32), jnp.uint32) >> 16
    hi16 = lax.bitcast_convert_type(
        hi.astype(jnp.bfloat16).astype(jnp.float32), jnp.uint32)
    hi16 = hi16 & jnp.uint32(0xFFFF0000)
    return lax.bitcast_convert_type(lo16 | hi16, jnp.float32)


def _unpack_bf16(p):
    u = lax.bitcast_convert_type(p, jnp.uint32)
    lo = lax.bitcast_convert_type(u << 16, jnp.float32)
    hi = lax.bitcast_convert_type(u & jnp.uint32(0xFFFF0000), jnp.float32)
    return lo, hi


def _edge_embed(edge_attr2d, edge_w_p, edge_b_p2d):
    BLK = 1280

    def body(a_ref, w_ref, b_ref, o_ref):
        a = a_ref[...]
        centers = lax.broadcasted_iota(
            jnp.int32, (1, NUM_BASIS), 1).astype(jnp.float32) * STEP
        diff = (a - centers) * (1.0 / STEP)
        basis = jnp.exp(-(diff * diff)) * (1.0 / 1.12)
        eaw = _matmul(basis, w_ref[...]) + b_ref[...]
        o_ref[...] = _pack_bf16(eaw[:, :_HALF], eaw[:, _HALF:])

    return pl.pallas_call(
        body,
        grid=(_E_PAD // BLK,),
        in_specs=[pl.BlockSpec((BLK, 1), lambda i: (i, 0)),
                  pl.BlockSpec((NUM_BASIS, D_MODEL), lambda i: (0, 0)),
                  pl.BlockSpec((1, D_MODEL), lambda i: (0, 0))],
        out_specs=pl.BlockSpec((BLK, _HALF), lambda i: (i, 0)),
        out_shape=jax.ShapeDtypeStruct((_E_PAD, _HALF), jnp.float32),
    )(edge_attr2d, edge_w_p, edge_b_p2d)


def _ln_qkv(h, lnw2d, lnb2d, w_qkv_p, b_qkv_p2d):
    BLK = 1000

    def body(h_ref, lw_ref, lb_ref, w_ref, b_ref, oq_ref, okv_ref):
        xh = h_ref[...]
        mu = jnp.mean(xh, axis=1, keepdims=True)
        xc = xh - mu
        var = jnp.mean(xc * xc, axis=1, keepdims=True)
        z = xc * lax.rsqrt(var + 1e-5) * lw_ref[...] + lb_ref[...]
        qkv = _matmul(z, w_ref[...]) + b_ref[...]
        qp = _pack_bf16(qkv[:, :_HALF], qkv[:, _HALF:D_MODEL])
        kp = _pack_bf16(qkv[:, D_MODEL:D_MODEL + _HALF],
                        qkv[:, D_MODEL + _HALF:2 * D_MODEL])
        vp = _pack_bf16(qkv[:, 2 * D_MODEL:2 * D_MODEL + _HALF],
                        qkv[:, 2 * D_MODEL + _HALF:])
        oq_ref[...] = qp
        okv_ref[...] = jnp.concatenate([kp, vp], axis=1)

    return pl.pallas_call(
        body,
        grid=(N // BLK,),
        in_specs=[pl.BlockSpec((BLK, D_MODEL), lambda i: (i, 0)),
                  pl.BlockSpec((1, D_MODEL), lambda i: (0, 0)),
                  pl.BlockSpec((1, D_MODEL), lambda i: (0, 0)),
                  pl.BlockSpec((D_MODEL, 3 * D_MODEL), lambda i: (0, 0)),
                  pl.BlockSpec((1, 3 * D_MODEL), lambda i: (0, 0))],
        out_specs=[pl.BlockSpec((BLK, _HALF), lambda i: (i, 0)),
                   pl.BlockSpec((BLK, D_MODEL), lambda i: (i, 0))],
        out_shape=[jax.ShapeDtypeStruct((N, _HALF), jnp.float32),
                   jax.ShapeDtypeStruct((N, D_MODEL), jnp.float32)],
    )(h, lnw2d, lnb2d, w_qkv_p, b_qkv_p2d)


def _edge_messages(qdp, kvp, ea):
    BLK = 640

    def body(qd_ref, kv_ref, ea_ref, oa_ref, ob_ref, ow_ref):
        qa, qb = _unpack_bf16(qd_ref[...])
        kv = kv_ref[...]
        ka, kb = _unpack_bf16(kv[:, :_HALF])
        va, vb = _unpack_bf16(kv[:, _HALF:])
        ea_a, ea_b = _unpack_bf16(ea_ref[...])
        ta = qa * (ka + ea_a)
        tb = qb * (kb + ea_b)
        logits = (_matmul(ta, _half_head_matrix()) +
                  _matmul(tb, _half_head_matrix())) * SCALE
        w = jnp.exp(logits)
        w_rep = _matmul(w, _head_half_matrix())
        oa_ref[...] = va * w_rep
        ob_ref[...] = vb * w_rep
        ow_ref[...] = jnp.concatenate(
            [w, jnp.zeros((BLK, _MSG_W - N_HEADS), jnp.float32)], axis=1)

    out = jax.ShapeDtypeStruct((_E_PAD, _MSG_W), jnp.float32)
    return pl.pallas_call(
        body,
        grid=(_E_PAD // BLK,),
        in_specs=[pl.BlockSpec((BLK, _HALF), lambda i: (i, 0)),
                  pl.BlockSpec((BLK, D_MODEL), lambda i: (i, 0)),
                  pl.BlockSpec((BLK, _HALF), lambda i: (i, 0))],
        out_specs=[pl.BlockSpec((BLK, _MSG_W), lambda i: (i, 0))] * 3,
        out_shape=[out, out, out],
    )(qdp, kvp, ea)


def _aggregate_update(h, pA, pB, pW, w_fc_p, b_fc2d):
    BLK = 1000

    def body(h_ref, pa_ref, pb_ref, pw_ref, w_ref, b_ref, o_ref):
        sA = pa_ref[0] + pa_ref[1]
        sB = pb_ref[0] + pb_ref[1]
        sW = pw_ref[0] + pw_ref[1]
        inv = 1.0 / (sW[:, :N_HEADS] + 1e-16)
        inv_rep = _matmul(inv, _head_split_matrix())
        agg = jnp.concatenate([sA, sB], axis=1) * inv_rep
        o_ref[...] = h_ref[...] + _matmul(agg, w_ref[...]) + b_ref[...]

    return pl.pallas_call(
        body,
        grid=(N // BLK,),
        in_specs=[pl.BlockSpec((BLK, D_MODEL), lambda i: (i, 0)),
                  pl.BlockSpec((_SC_CORES, BLK, _MSG_W), lambda i: (0, i, 0)),
                  pl.BlockSpec((_SC_CORES, BLK, _MSG_W), lambda i: (0, i, 0)),
                  pl.BlockSpec((_SC_CORES, BLK, _MSG_W), lambda i: (0, i, 0)),
                  pl.BlockSpec((D_MODEL, D_MODEL), lambda i: (0, 0)),
                  pl.BlockSpec((1, D_MODEL), lambda i: (0, 0))],
        out_specs=pl.BlockSpec((BLK, D_MODEL), lambda i: (i, 0)),
        out_shape=jax.ShapeDtypeStruct((N, D_MODEL), jnp.float32),
    )(h, pA, pB, pW, w_fc_p, b_fc2d)


def _sc_mesh():
    return plsc.VectorSubcoreMesh(core_axis_name="c", subcore_axis_name="s")


def _sc_gather_rows(table, idx2d):
    m = idx2d.shape[1]
    row = table.shape[1]

    @functools.partial(
        pl.kernel,
        out_type=jax.ShapeDtypeStruct((m, row), table.dtype),
        mesh=_sc_mesh())
    def kern(t_hbm, i_hbm, o_hbm):
        def body(i_vmem, o_vmem):
            pltpu.sync_copy(t_hbm.at[i_vmem.at[0]], o_vmem)

        pltpu.emit_pipeline(
            body,
            grid=(m // _WINDOW,),
            in_specs=[pl.BlockSpec((1, _WINDOW), lambda i: (0, i))],
            out_specs=[pl.BlockSpec((_WINDOW, row), lambda i: (i, 0))],
            core_axis_name=("c", "s"),
            dimension_semantics=(pltpu.PARALLEL,),
        )(i_hbm, o_hbm)

    return kern(table, idx2d)


def _sc_gather_qkv(qp, kvp, dst2d, src2d):
    @functools.partial(
        pl.kernel,
        out_type=(jax.ShapeDtypeStruct((_E_PAD, _HALF), jnp.float32),
                  jax.ShapeDtypeStruct((_E_PAD, D_MODEL), jnp.float32)),
        mesh=_sc_mesh())
    def kern(q_hbm, kv_hbm, d_hbm, s_hbm, oq_hbm, okv_hbm):
        def body(d_vmem, s_vmem, oq_vmem, okv_vmem):
            pltpu.sync_copy(q_hbm.at[d_vmem.at[0]], oq_vmem)
            pltpu.sync_copy(kv_hbm.at[s_vmem.at[0]], okv_vmem)

        pltpu.emit_pipeline(
            body,
            grid=(_E_PAD // _WINDOW,),
            in_specs=[pl.BlockSpec((1, _WINDOW), lambda i: (0, i)),
                      pl.BlockSpec((1, _WINDOW), lambda i: (0, i))],
            out_specs=[pl.BlockSpec((_WINDOW, _HALF), lambda i: (i, 0)),
                       pl.BlockSpec((_WINDOW, D_MODEL), lambda i: (i, 0))],
            core_axis_name=("c", "s"),
            dimension_semantics=(pltpu.PARALLEL,),
        )(d_hbm, s_hbm, oq_hbm, okv_hbm)

    return kern(qp, kvp, dst2d, src2d)


def _sc_scatter_add3(valsA, valsB, valsW, idx2d, zeros_sub):
    out = jax.ShapeDtypeStruct((_SC_CORES, _ACC_ROWS, _MSG_W), jnp.float32)

    @functools.partial(
        pl.kernel,
        out_type=(out, out, out),
        mesh=_sc_mesh(),
        scratch_types=[pltpu.VMEM_SHARED((_ACC_ROWS, _MSG_W), jnp.float32)])
    def kern(va_hbm, vb_hbm, vw_hbm, i_hbm, z_hbm, oa_hbm, ob_hbm, ow_hbm,
             acc):
        cid = lax.axis_index("c")
        sid = lax.axis_index("s")
        row0 = sid * _ROWS_PER_SUBCORE

        def body(v_vmem, i_vmem):
            pltpu.sync_copy(v_vmem, acc.at[i_vmem.at[0]], add=True)

        def one_pass(v_hbm, o_hbm):
            pltpu.sync_copy(z_hbm, acc.at[pl.ds(row0, _ROWS_PER_SUBCORE)])
            plsc.subcore_barrier()
            pltpu.emit_pipeline(
                body,
                grid=(_E_PAD // _WINDOW,),
                in_specs=[pl.BlockSpec((_WINDOW, _MSG_W), lambda i: (i, 0)),
                          pl.BlockSpec((1, _WINDOW), lambda i: (0, i))],
                out_specs=[],
                core_axis_name=("c", "s"),
                dimension_semantics=(pltpu.PARALLEL,),
            )(v_hbm, i_hbm)
            plsc.subcore_barrier()
            pltpu.sync_copy(acc.at[pl.ds(row0, _ROWS_PER_SUBCORE)],
                            o_hbm.at[cid, pl.ds(row0, _ROWS_PER_SUBCORE)])
            plsc.subcore_barrier()

        one_pass(va_hbm, oa_hbm)
        one_pass(vb_hbm, ob_hbm)
        one_pass(vw_hbm, ow_hbm)

    return kern(valsA, valsB, valsW, idx2d, zeros_sub)


def _sc_scatter_add(vals, idx2d, zeros_sub):
    @functools.partial(
        pl.kernel,
        out_type=jax.ShapeDtypeStruct((_SC_CORES, _ACC_ROWS, _MSG_W),
                                      jnp.float32),
        mesh=_sc_mesh(),
        scratch_types=[pltpu.VMEM_SHARED((_ACC_ROWS, _MSG_W), jnp.float32)])
    def kern(v_hbm, i_hbm, z_hbm, o_hbm, acc):
        cid = lax.axis_index("c")
        sid = lax.axis_index("s")
        row0 = sid * _ROWS_PER_SUBCORE
        pltpu.sync_copy(z_hbm, acc.at[pl.ds(row0, _ROWS_PER_SUBCORE)])
        plsc.subcore_barrier()

        def body(v_vmem, i_vmem):
            pltpu.sync_copy(v_vmem, acc.at[i_vmem.at[0]], add=True)

        pltpu.emit_pipeline(
            body,
            grid=(E // _WINDOW,),
            in_specs=[pl.BlockSpec((_WINDOW, _MSG_W), lambda i: (i, 0)),
                      pl.BlockSpec((1, _WINDOW), lambda i: (0, i))],
            out_specs=[],
            core_axis_name=("c", "s"),
            dimension_semantics=(pltpu.PARALLEL,),
        )(v_hbm, i_hbm)

        plsc.subcore_barrier()
        pltpu.sync_copy(acc.at[pl.ds(row0, _ROWS_PER_SUBCORE)],
                        o_hbm.at[cid, pl.ds(row0, _ROWS_PER_SUBCORE)])

    return kern(vals, idx2d, zeros_sub)


def kernel(x, edge_index, edge_attr, atom_table, edge_w, edge_b,
           qkv_w, qkv_b, fc_w, fc_b, ln_w, ln_b):
    pad = jnp.zeros((_E_PAD - E,), jnp.int32)
    src2d = jnp.concatenate([edge_index[0], pad]).reshape(1, _E_PAD)
    dst2d = jnp.concatenate([edge_index[1], pad]).reshape(1, _E_PAD)
    trash = N + jnp.arange(_E_PAD - E, dtype=jnp.int32) % (_ACC_ROWS - N)
    dst_s = jnp.concatenate([edge_index[1], trash]).reshape(1, _E_PAD)

    perm = jnp.asarray(_P)
    perm_qkv = jnp.concatenate([perm, perm + D_MODEL, perm + 2 * D_MODEL])
    ea = _edge_embed(
        jnp.concatenate([edge_attr, jnp.zeros((_E_PAD - E,),
                                              jnp.float32)]).reshape(_E_PAD, 1),
        edge_w[:, perm], edge_b[perm].reshape(1, D_MODEL))

    table_scaled = atom_table * math.sqrt(D_MODEL)
    x_pad = jnp.concatenate(
        [x, jnp.zeros((_EMB_PAD - N,), jnp.int32)]).reshape(1, _EMB_PAD)
    h = _sc_gather_rows(table_scaled, x_pad)[:N]

    zeros_sub = jnp.zeros((_ROWS_PER_SUBCORE, _MSG_W), jnp.float32)
    for l in range(N_LAYERS):
        qp, kvp = _ln_qkv(h, ln_w[l].reshape(1, -1), ln_b[l].reshape(1, -1),
                          qkv_w[l][:, perm_qkv],
                          qkv_b[l][perm_qkv].reshape(1, -1))
        qdp, kvs = _sc_gather_qkv(qp, kvp, dst2d, src2d)
        msgA, msgB, msgW = _edge_messages(qdp, kvs, ea)
        pA, pB, pW = _sc_scatter_add3(msgA, msgB, msgW, dst_s, zeros_sub)
        h = _aggregate_update(h, pA, pB, pW, fc_w[l][perm, :],
                              fc_b[l].reshape(1, -1))
    return h


# speedup vs baseline: 1.4215x; 1.4215x over previous
import dataclasses
import functools
import math

import jax
import jax.numpy as jnp
import numpy as np
from jax import lax
from jax.experimental import pallas as pl
from jax.experimental.pallas import tpu as pltpu
from jax.experimental.pallas import tpu_sc as plsc

N = 10000
E = 160000
D_MODEL = 256
N_HEADS = 8
D_HEAD = D_MODEL // N_HEADS
N_LAYERS = 2
CUTOFF = 8.0
STEP = 0.2
NUM_BASIS = int(CUTOFF / STEP) + 1
SCALE = D_HEAD ** (-0.5)

_SC_SUBCORES = 16
_SC_CORES = 2
_ACC_ROWS = 10240
_ROWS_PER_SUBCORE = _ACC_ROWS // _SC_SUBCORES
_WINDOW = 128
_MSG_W = 128
_EMB_PAD = 10240
_E_PAD = 163840
_STEPS_PER_TILE = _E_PAD // 32 // _WINDOW
_HALF = D_MODEL // 2

_HIGHEST = lax.Precision.HIGHEST

_P = np.array([(i % _HALF // 16) * D_HEAD + (i // _HALF) * (D_HEAD // 2)
               + i % 16 for i in range(D_MODEL)], dtype=np.int32)


def _matmul(a, b):
    return lax.dot_general(a, b, (((1,), (0,)), ((), ())),
                           precision=_HIGHEST,
                           preferred_element_type=jnp.float32)


def _half_head_matrix():
    lane = lax.broadcasted_iota(jnp.int32, (_HALF, N_HEADS), 0)
    head = lax.broadcasted_iota(jnp.int32, (_HALF, N_HEADS), 1)
    return (lane // 16 == head).astype(jnp.float32)


def _head_half_matrix():
    lane = lax.broadcasted_iota(jnp.int32, (N_HEADS, _HALF), 1)
    head = lax.broadcasted_iota(jnp.int32, (N_HEADS, _HALF), 0)
    return (lane // 16 == head).astype(jnp.float32)


def _head_split_matrix():
    lane = lax.broadcasted_iota(jnp.int32, (N_HEADS, D_MODEL), 1)
    head = lax.broadcasted_iota(jnp.int32, (N_HEADS, D_MODEL), 0)
    return (lane % _HALF // 16 == head).astype(jnp.float32)


def _pack_bf16(lo, hi):
    lo16 = lax.bitcast_convert_type(
        lo.astype(jnp.bfloat16).astype(jnp.float32), jnp.uint32) >> 16
    hi16 = lax.bitcast_convert_type(
        hi.astype(jnp.bfloat16).astype(jnp.float32), jnp.uint32)
    hi16 = hi16 & jnp.uint32(0xFFFF0000)
    return lax.bitcast_convert_type(lo16 | hi16, jnp.float32)


def _unpack_bf16(p):
    u = lax.bitcast_convert_type(p, jnp.uint32)
    lo = lax.bitcast_convert_type(u << 16, jnp.float32)
    hi = lax.bitcast_convert_type(u & jnp.uint32(0xFFFF0000), jnp.float32)
    return lo, hi


def _edge_embed(edge_attr2d, edge_w_p, edge_b_p2d):
    BLK = 1280

    def body(a_ref, w_ref, b_ref, o_ref):
        a = a_ref[...]
        centers = lax.broadcasted_iota(
            jnp.int32, (1, NUM_BASIS), 1).astype(jnp.float32) * STEP
        diff = (a - centers) * (1.0 / STEP)
        basis = jnp.exp(-(diff * diff)) * (1.0 / 1.12)
        eaw = _matmul(basis, w_ref[...]) + b_ref[...]
        o_ref[...] = _pack_bf16(eaw[:, :_HALF], eaw[:, _HALF:])

    return pl.pallas_call(
        body,
        grid=(_E_PAD // BLK,),
        in_specs=[pl.BlockSpec((BLK, 1), lambda i: (i, 0)),
                  pl.BlockSpec((NUM_BASIS, D_MODEL), lambda i: (0, 0)),
                  pl.BlockSpec((1, D_MODEL), lambda i: (0, 0))],
        out_specs=pl.BlockSpec((BLK, _HALF), lambda i: (i, 0)),
        out_shape=jax.ShapeDtypeStruct((_E_PAD, _HALF), jnp.float32),
    )(edge_attr2d, edge_w_p, edge_b_p2d)


def _ln_qkv(h, lnw2d, lnb2d, w_qkv_p, b_qkv_p2d):
    BLK = 1000

    def body(h_ref, lw_ref, lb_ref, w_ref, b_ref, oq_ref, okv_ref):
        xh = h_ref[...]
        mu = jnp.mean(xh, axis=1, keepdims=True)
        xc = xh - mu
        var = jnp.mean(xc * xc, axis=1, keepdims=True)
        z = xc * lax.rsqrt(var + 1e-5) * lw_ref[...] + lb_ref[...]
        qkv = _matmul(z, w_ref[...]) + b_ref[...]
        qp = _pack_bf16(qkv[:, :_HALF], qkv[:, _HALF:D_MODEL])
        kp = _pack_bf16(qkv[:, D_MODEL:D_MODEL + _HALF],
                        qkv[:, D_MODEL + _HALF:2 * D_MODEL])
        vp = _pack_bf16(qkv[:, 2 * D_MODEL:2 * D_MODEL + _HALF],
                        qkv[:, 2 * D_MODEL + _HALF:])
        oq_ref[...] = qp
        okv_ref[...] = jnp.concatenate([kp, vp], axis=1)

    return pl.pallas_call(
        body,
        grid=(N // BLK,),
        in_specs=[pl.BlockSpec((BLK, D_MODEL), lambda i: (i, 0)),
                  pl.BlockSpec((1, D_MODEL), lambda i: (0, 0)),
                  pl.BlockSpec((1, D_MODEL), lambda i: (0, 0)),
                  pl.BlockSpec((D_MODEL, 3 * D_MODEL), lambda i: (0, 0)),
                  pl.BlockSpec((1, 3 * D_MODEL), lambda i: (0, 0))],
        out_specs=[pl.BlockSpec((BLK, _HALF), lambda i: (i, 0)),
                   pl.BlockSpec((BLK, D_MODEL), lambda i: (i, 0))],
        out_shape=[jax.ShapeDtypeStruct((N, _HALF), jnp.float32),
                   jax.ShapeDtypeStruct((N, D_MODEL), jnp.float32)],
    )(h, lnw2d, lnb2d, w_qkv_p, b_qkv_p2d)


def _edge_messages(qdp, kvp, ea):
    BLK = 640

    def body(qd_ref, kv_ref, ea_ref, oa_ref, ob_ref, ow_ref):
        qa, qb = _unpack_bf16(qd_ref[...])
        kv = kv_ref[...]
        ka, kb = _unpack_bf16(kv[:, :_HALF])
        va, vb = _unpack_bf16(kv[:, _HALF:])
        ea_a, ea_b = _unpack_bf16(ea_ref[...])
        ta = qa * (ka + ea_a)
        tb = qb * (kb + ea_b)
        logits = (_matmul(ta, _half_head_matrix()) +
                  _matmul(tb, _half_head_matrix())) * SCALE
        w = jnp.exp(logits)
        w_rep = _matmul(w, _head_half_matrix())
        oa_ref[...] = va * w_rep
        ob_ref[...] = vb * w_rep
        ow_ref[...] = jnp.concatenate(
            [w, jnp.zeros((BLK, _MSG_W - N_HEADS), jnp.float32)], axis=1)

    out = jax.ShapeDtypeStruct((_E_PAD, _MSG_W), jnp.float32)
    return pl.pallas_call(
        body,
        grid=(_E_PAD // BLK,),
        in_specs=[pl.BlockSpec((BLK, _HALF), lambda i: (i, 0)),
                  pl.BlockSpec((BLK, D_MODEL), lambda i: (i, 0)),
                  pl.BlockSpec((BLK, _HALF), lambda i: (i, 0))],
        out_specs=[pl.BlockSpec((BLK, _MSG_W), lambda i: (i, 0))] * 3,
        out_shape=[out, out, out],
    )(qdp, kvp, ea)


def _aggregate_update(h, pA, pB, pW, w_fc_p, b_fc2d):
    BLK = 1000

    def body(h_ref, pa_ref, pb_ref, pw_ref, w_ref, b_ref, o_ref):
        sA = pa_ref[0] + pa_ref[1]
        sB = pb_ref[0] + pb_ref[1]
        sW = pw_ref[0] + pw_ref[1]
        inv = 1.0 / (sW[:, :N_HEADS] + 1e-16)
        inv_rep = _matmul(inv, _head_split_matrix())
        agg = jnp.concatenate([sA, sB], axis=1) * inv_rep
        o_ref[...] = h_ref[...] + _matmul(agg, w_ref[...]) + b_ref[...]

    return pl.pallas_call(
        body,
        grid=(N // BLK,),
        in_specs=[pl.BlockSpec((BLK, D_MODEL), lambda i: (i, 0)),
                  pl.BlockSpec((_SC_CORES, BLK, _MSG_W), lambda i: (0, i, 0)),
                  pl.BlockSpec((_SC_CORES, BLK, _MSG_W), lambda i: (0, i, 0)),
                  pl.BlockSpec((_SC_CORES, BLK, _MSG_W), lambda i: (0, i, 0)),
                  pl.BlockSpec((D_MODEL, D_MODEL), lambda i: (0, 0)),
                  pl.BlockSpec((1, D_MODEL), lambda i: (0, 0))],
        out_specs=pl.BlockSpec((BLK, D_MODEL), lambda i: (i, 0)),
        out_shape=jax.ShapeDtypeStruct((N, D_MODEL), jnp.float32),
    )(h, pA, pB, pW, w_fc_p, b_fc2d)


def _sc_mesh():
    return plsc.VectorSubcoreMesh(core_axis_name="c", subcore_axis_name="s")


def _sc_gather_rows(table, idx2d):
    m = idx2d.shape[1]
    row = table.shape[1]

    @functools.partial(
        pl.kernel,
        out_type=jax.ShapeDtypeStruct((m, row), table.dtype),
        mesh=_sc_mesh())
    def kern(t_hbm, i_hbm, o_hbm):
        def body(i_vmem, o_vmem):
            pltpu.sync_copy(t_hbm.at[i_vmem.at[0]], o_vmem)

        pltpu.emit_pipeline(
            body,
            grid=(m // _WINDOW,),
            in_specs=[pl.BlockSpec((1, _WINDOW), lambda i: (0, i))],
            out_specs=[pl.BlockSpec((_WINDOW, row), lambda i: (i, 0))],
            core_axis_name=("c", "s"),
            dimension_semantics=(pltpu.PARALLEL,),
        )(i_hbm, o_hbm)

    return kern(table, idx2d)


def _sc_gather_qkv(qp, kvp, dst2d, src2d):
    @functools.partial(
        pl.kernel,
        out_type=(jax.ShapeDtypeStruct((_E_PAD, _HALF), jnp.float32),
                  jax.ShapeDtypeStruct((_E_PAD, D_MODEL), jnp.float32)),
        mesh=_sc_mesh())
    def kern(q_hbm, kv_hbm, d_hbm, s_hbm, oq_hbm, okv_hbm):
        def body(d_vmem, s_vmem, oq_vmem, okv_vmem):
            pltpu.sync_copy(q_hbm.at[d_vmem.at[0]], oq_vmem)
            pltpu.sync_copy(kv_hbm.at[s_vmem.at[0]], okv_vmem)

        pltpu.emit_pipeline(
            body,
            grid=(_E_PAD // _WINDOW,),
            in_specs=[pl.BlockSpec((1, _WINDOW), lambda i: (0, i)),
                      pl.BlockSpec((1, _WINDOW), lambda i: (0, i))],
            out_specs=[pl.BlockSpec((_WINDOW, _HALF), lambda i: (i, 0)),
                       pl.BlockSpec((_WINDOW, D_MODEL), lambda i: (i, 0))],
            core_axis_name=("c", "s"),
            dimension_semantics=(pltpu.PARALLEL,),
        )(d_hbm, s_hbm, oq_hbm, okv_hbm)

    return kern(qp, kvp, dst2d, src2d)


def _sc_scatter_add3(valsA, valsB, valsW, idx2d, zeros_sub):
    out = jax.ShapeDtypeStruct((_SC_CORES, _ACC_ROWS, _MSG_W), jnp.float32)

    @functools.partial(
        pl.kernel,
        out_type=(out, out, out),
        mesh=_sc_mesh(),
        scratch_types=[pltpu.VMEM_SHARED((_ACC_ROWS, _MSG_W), jnp.float32)])
    def kern(va_hbm, vb_hbm, vw_hbm, i_hbm, z_hbm, oa_hbm, ob_hbm, ow_hbm,
             acc):
        cid = lax.axis_index("c")
        sid = lax.axis_index("s")
        row0 = sid * _ROWS_PER_SUBCORE

        def body(v_vmem, i_vmem):
            pltpu.sync_copy(v_vmem, acc.at[i_vmem.at[0]], add=True)

        def one_pass(v_hbm, o_hbm):
            pltpu.sync_copy(z_hbm, acc.at[pl.ds(row0, _ROWS_PER_SUBCORE)])
            plsc.subcore_barrier()
            pltpu.emit_pipeline(
                body,
                grid=(_E_PAD // _WINDOW,),
                in_specs=[pl.BlockSpec((_WINDOW, _MSG_W), lambda i: (i, 0)),
                          pl.BlockSpec((1, _WINDOW), lambda i: (0, i))],
                out_specs=[],
                core_axis_name=("c", "s"),
                dimension_semantics=(pltpu.PARALLEL,),
            )(v_hbm, i_hbm)
            plsc.subcore_barrier()
            pltpu.sync_copy(acc.at[pl.ds(row0, _ROWS_PER_SUBCORE)],
                            o_hbm.at[cid, pl.ds(row0, _ROWS_PER_SUBCORE)])
            plsc.subcore_barrier()

        one_pass(va_hbm, oa_hbm)
        one_pass(vb_hbm, ob_hbm)
        one_pass(vw_hbm, ow_hbm)

    return kern(valsA, valsB, valsW, idx2d, zeros_sub)


def _sc_scatter_add(vals, idx2d, zeros_sub):
    @functools.partial(
        pl.kernel,
        out_type=jax.ShapeDtypeStruct((_SC_CORES, _ACC_ROWS, _MSG_W),
                                      jnp.float32),
        mesh=_sc_mesh(),
        scratch_types=[pltpu.VMEM_SHARED((_ACC_ROWS, _MSG_W), jnp.float32)])
    def kern(v_hbm, i_hbm, z_hbm, o_hbm, acc):
        cid = lax.axis_index("c")
        sid = lax.axis_index("s")
        row0 = sid * _ROWS_PER_SUBCORE
        pltpu.sync_copy(z_hbm, acc.at[pl.ds(row0, _ROWS_PER_SUBCORE)])
        plsc.subcore_barrier()

        def body(v_vmem, i_vmem):
            pltpu.sync_copy(v_vmem, acc.at[i_vmem.at[0]], add=True)

        pltpu.emit_pipeline(
            body,
            grid=(E // _WINDOW,),
            in_specs=[pl.BlockSpec((_WINDOW, _MSG_W), lambda i: (i, 0)),
                      pl.BlockSpec((1, _WINDOW), lambda i: (0, i))],
            out_specs=[],
            core_axis_name=("c", "s"),
            dimension_semantics=(pltpu.PARALLEL,),
        )(v_hbm, i_hbm)

        plsc.subcore_barrier()
        pltpu.sync_copy(acc.at[pl.ds(row0, _ROWS_PER_SUBCORE)],
                        o_hbm.at[cid, pl.ds(row0, _ROWS_PER_SUBCORE)])

    return kern(vals, idx2d, zeros_sub)


def kernel(x, edge_index, edge_attr, atom_table, edge_w, edge_b,
           qkv_w, qkv_b, fc_w, fc_b, ln_w, ln_b):
    pad = jnp.arange(_E_PAD - E, dtype=jnp.int32) % N
    src2d = jnp.concatenate([edge_index[0], pad]).reshape(1, _E_PAD)
    dst2d = jnp.concatenate([edge_index[1], pad]).reshape(1, _E_PAD)
    trash = N + jnp.arange(_E_PAD - E, dtype=jnp.int32) % (_ACC_ROWS - N)
    dst_s = jnp.concatenate([edge_index[1], trash]).reshape(1, _E_PAD)


    perm = jnp.asarray(_P)
    perm_qkv = jnp.concatenate([perm, perm + D_MODEL, perm + 2 * D_MODEL])
    ea = _edge_embed(
        jnp.concatenate([edge_attr, jnp.zeros((_E_PAD - E,),
                                              jnp.float32)]).reshape(_E_PAD, 1),
        edge_w[:, perm], edge_b[perm].reshape(1, D_MODEL))

    table_scaled = atom_table * math.sqrt(D_MODEL)
    x_pad = jnp.concatenate(
        [x, jnp.zeros((_EMB_PAD - N,), jnp.int32)]).reshape(1, _EMB_PAD)
    h = _sc_gather_rows(table_scaled, x_pad)[:N]

    zeros_sub = jnp.zeros((_ROWS_PER_SUBCORE, _MSG_W), jnp.float32)
    for l in range(N_LAYERS):
        qp, kvp = _ln_qkv(h, ln_w[l].reshape(1, -1), ln_b[l].reshape(1, -1),
                          qkv_w[l][:, perm_qkv],
                          qkv_b[l][perm_qkv].reshape(1, -1))
        qdp, kvs = _sc_gather_qkv(qp, kvp, dst2d, src2d)
        msgA, msgB, msgW = _edge_messages(qdp, kvs, ea)
        pA, pB, pW = _sc_scatter_add3(msgA, msgB, msgW, dst_s, zeros_sub)
        h = _aggregate_update(h, pA, pB, pW, fc_w[l][perm, :],
                              fc_b[l].reshape(1, -1))
    return h


# speedup vs baseline: 1.4838x; 1.0438x over previous
import dataclasses
import functools
import math

import jax
import jax.numpy as jnp
import numpy as np
from jax import lax
from jax.experimental import pallas as pl
from jax.experimental.pallas import tpu as pltpu
from jax.experimental.pallas import tpu_sc as plsc

N = 10000
E = 160000
D_MODEL = 256
N_HEADS = 8
D_HEAD = D_MODEL // N_HEADS
N_LAYERS = 2
CUTOFF = 8.0
STEP = 0.2
NUM_BASIS = int(CUTOFF / STEP) + 1
SCALE = D_HEAD ** (-0.5)

_SC_SUBCORES = 16
_SC_CORES = 2
_ACC_ROWS = 10240
_ROWS_PER_SUBCORE = _ACC_ROWS // _SC_SUBCORES
_WINDOW = 128
_MSG_W = 128
_EMB_PAD = 10240
_E_PAD = 163840
_NCHUNK = 2
_E_CHUNK = _E_PAD // _NCHUNK
_HALF = D_MODEL // 2

_HIGHEST = lax.Precision.HIGHEST

_P = np.array([(i % _HALF // 16) * D_HEAD + (i // _HALF) * (D_HEAD // 2)
               + i % 16 for i in range(D_MODEL)], dtype=np.int32)


def _matmul(a, b):
    return lax.dot_general(a, b, (((1,), (0,)), ((), ())),
                           precision=_HIGHEST,
                           preferred_element_type=jnp.float32)


def _half_head_matrix():
    lane = lax.broadcasted_iota(jnp.int32, (_HALF, N_HEADS), 0)
    head = lax.broadcasted_iota(jnp.int32, (_HALF, N_HEADS), 1)
    return (lane // 16 == head).astype(jnp.float32)


def _head_half_matrix():
    lane = lax.broadcasted_iota(jnp.int32, (N_HEADS, _HALF), 1)
    head = lax.broadcasted_iota(jnp.int32, (N_HEADS, _HALF), 0)
    return (lane // 16 == head).astype(jnp.float32)


def _head_split_matrix():
    lane = lax.broadcasted_iota(jnp.int32, (N_HEADS, D_MODEL), 1)
    head = lax.broadcasted_iota(jnp.int32, (N_HEADS, D_MODEL), 0)
    return (lane % _HALF // 16 == head).astype(jnp.float32)


def _pack_bf16(lo, hi):
    lo16 = lax.bitcast_convert_type(
        lo.astype(jnp.bfloat16).astype(jnp.float32), jnp.uint32) >> 16
    hi16 = lax.bitcast_convert_type(
        hi.astype(jnp.bfloat16).astype(jnp.float32), jnp.uint32)
    hi16 = hi16 & jnp.uint32(0xFFFF0000)
    return lax.bitcast_convert_type(lo16 | hi16, jnp.float32)


def _unpack_bf16(p):
    u = lax.bitcast_convert_type(p, jnp.uint32)
    lo = lax.bitcast_convert_type(u << 16, jnp.float32)
    hi = lax.bitcast_convert_type(u & jnp.uint32(0xFFFF0000), jnp.float32)
    return lo, hi


def _edge_embed(edge_attr2d, edge_w_p, edge_b_p2d):
    BLK = 1280

    def body(a_ref, w_ref, b_ref, o_ref):
        a = a_ref[...]
        centers = lax.broadcasted_iota(
            jnp.int32, (1, NUM_BASIS), 1).astype(jnp.float32) * STEP
        diff = (a - centers) * (1.0 / STEP)
        basis = jnp.exp(-(diff * diff)) * (1.0 / 1.12)
        eaw = _matmul(basis, w_ref[...]) + b_ref[...]
        o_ref[...] = _pack_bf16(eaw[:, :_HALF], eaw[:, _HALF:])

    return pl.pallas_call(
        body,
        grid=(_E_PAD // BLK,),
        in_specs=[pl.BlockSpec((BLK, 1), lambda i: (i, 0)),
                  pl.BlockSpec((NUM_BASIS, D_MODEL), lambda i: (0, 0)),
                  pl.BlockSpec((1, D_MODEL), lambda i: (0, 0))],
        out_specs=pl.BlockSpec((BLK, _HALF), lambda i: (i, 0)),
        out_shape=jax.ShapeDtypeStruct((_E_PAD, _HALF), jnp.float32),
    )(edge_attr2d, edge_w_p, edge_b_p2d)


def _ln_qkv(h, lnw2d, lnb2d, w_qkv_p, b_qkv_p2d):
    BLK = 1000

    def body(h_ref, lw_ref, lb_ref, w_ref, b_ref, oq_ref, okv_ref):
        xh = h_ref[...]
        mu = jnp.mean(xh, axis=1, keepdims=True)
        xc = xh - mu
        var = jnp.mean(xc * xc, axis=1, keepdims=True)
        z = xc * lax.rsqrt(var + 1e-5) * lw_ref[...] + lb_ref[...]
        qkv = _matmul(z, w_ref[...]) + b_ref[...]
        qp = _pack_bf16(qkv[:, :_HALF], qkv[:, _HALF:D_MODEL])
        kp = _pack_bf16(qkv[:, D_MODEL:D_MODEL + _HALF],
                        qkv[:, D_MODEL + _HALF:2 * D_MODEL])
        vp = _pack_bf16(qkv[:, 2 * D_MODEL:2 * D_MODEL + _HALF],
                        qkv[:, 2 * D_MODEL + _HALF:])
        oq_ref[...] = qp
        okv_ref[...] = jnp.concatenate([kp, vp], axis=1)

    return pl.pallas_call(
        body,
        grid=(N // BLK,),
        in_specs=[pl.BlockSpec((BLK, D_MODEL), lambda i: (i, 0)),
                  pl.BlockSpec((1, D_MODEL), lambda i: (0, 0)),
                  pl.BlockSpec((1, D_MODEL), lambda i: (0, 0)),
                  pl.BlockSpec((D_MODEL, 3 * D_MODEL), lambda i: (0, 0)),
                  pl.BlockSpec((1, 3 * D_MODEL), lambda i: (0, 0))],
        out_specs=[pl.BlockSpec((BLK, _HALF), lambda i: (i, 0)),
                   pl.BlockSpec((BLK, D_MODEL), lambda i: (i, 0))],
        out_shape=[jax.ShapeDtypeStruct((N, _HALF), jnp.float32),
                   jax.ShapeDtypeStruct((N, D_MODEL), jnp.float32)],
    )(h, lnw2d, lnb2d, w_qkv_p, b_qkv_p2d)


def _edge_messages(qdp, kvp, ea, c):
    BLK = 640
    off = c * (_E_CHUNK // BLK)

    def body(qd_ref, kv_ref, ea_ref, oa_ref, ob_ref, ow_ref):
        qa, qb = _unpack_bf16(qd_ref[...])
        kv = kv_ref[...]
        ka, kb = _unpack_bf16(kv[:, :_HALF])
        va, vb = _unpack_bf16(kv[:, _HALF:])
        ea_a, ea_b = _unpack_bf16(ea_ref[...])
        ta = qa * (ka + ea_a)
        tb = qb * (kb + ea_b)
        logits = (_matmul(ta, _half_head_matrix()) +
                  _matmul(tb, _half_head_matrix())) * SCALE
        w = jnp.exp(logits)
        w_rep = _matmul(w, _head_half_matrix())
        oa_ref[...] = va * w_rep
        ob_ref[...] = vb * w_rep
        ow_ref[...] = jnp.concatenate(
            [w, jnp.zeros((BLK, _MSG_W - N_HEADS), jnp.float32)], axis=1)

    out = jax.ShapeDtypeStruct((_E_CHUNK, _MSG_W), jnp.float32)
    return pl.pallas_call(
        body,
        grid=(_E_CHUNK // BLK,),
        in_specs=[pl.BlockSpec((BLK, _HALF), lambda i: (i + off, 0)),
                  pl.BlockSpec((BLK, D_MODEL), lambda i: (i + off, 0)),
                  pl.BlockSpec((BLK, _HALF), lambda i: (i + off, 0))],
        out_specs=[pl.BlockSpec((BLK, _MSG_W), lambda i: (i, 0))] * 3,
        out_shape=[out, out, out],
    )(qdp, kvp, ea)


def _aggregate_update(h, parts, w_fc_p, b_fc2d):
    BLK = 1000

    def body(h_ref, pa0_ref, pb0_ref, pw0_ref, pa1_ref, pb1_ref, pw1_ref,
             w_ref, b_ref, o_ref):
        sA = pa0_ref[0] + pa0_ref[1] + pa1_ref[0] + pa1_ref[1]
        sB = pb0_ref[0] + pb0_ref[1] + pb1_ref[0] + pb1_ref[1]
        sW = pw0_ref[0] + pw0_ref[1] + pw1_ref[0] + pw1_ref[1]
        inv = 1.0 / (sW[:, :N_HEADS] + 1e-16)
        inv_rep = _matmul(inv, _head_split_matrix())
        agg = jnp.concatenate([sA, sB], axis=1) * inv_rep
        o_ref[...] = h_ref[...] + _matmul(agg, w_ref[...]) + b_ref[...]

    return pl.pallas_call(
        body,
        grid=(N // BLK,),
        in_specs=[pl.BlockSpec((BLK, D_MODEL), lambda i: (i, 0))] +
                 [pl.BlockSpec((_SC_CORES, BLK, _MSG_W),
                               lambda i: (0, i, 0))] * 6 +
                 [pl.BlockSpec((D_MODEL, D_MODEL), lambda i: (0, 0)),
                  pl.BlockSpec((1, D_MODEL), lambda i: (0, 0))],
        out_specs=pl.BlockSpec((BLK, D_MODEL), lambda i: (i, 0)),
        out_shape=jax.ShapeDtypeStruct((N, D_MODEL), jnp.float32),
    )(h, *parts, w_fc_p, b_fc2d)


def _sc_mesh():
    return plsc.VectorSubcoreMesh(core_axis_name="c", subcore_axis_name="s")


def _sc_gather_rows(table, idx2d):
    m = idx2d.shape[1]
    row = table.shape[1]

    @functools.partial(
        pl.kernel,
        out_type=jax.ShapeDtypeStruct((m, row), table.dtype),
        mesh=_sc_mesh())
    def kern(t_hbm, i_hbm, o_hbm):
        def body(i_vmem, o_vmem):
            pltpu.sync_copy(t_hbm.at[i_vmem.at[0]], o_vmem)

        pltpu.emit_pipeline(
            body,
            grid=(m // _WINDOW,),
            in_specs=[pl.BlockSpec((1, _WINDOW), lambda i: (0, i))],
            out_specs=[pl.BlockSpec((_WINDOW, row), lambda i: (i, 0))],
            core_axis_name=("c", "s"),
            dimension_semantics=(pltpu.PARALLEL,),
        )(i_hbm, o_hbm)

    return kern(table, idx2d)


def _sc_gather_qkv(qp, kvp, dst2d, src2d):
    @functools.partial(
        pl.kernel,
        out_type=(jax.ShapeDtypeStruct((_E_PAD, _HALF), jnp.float32),
                  jax.ShapeDtypeStruct((_E_PAD, D_MODEL), jnp.float32)),
        mesh=_sc_mesh())
    def kern(q_hbm, kv_hbm, d_hbm, s_hbm, oq_hbm, okv_hbm):
        def body(d_vmem, s_vmem, oq_vmem, okv_vmem):
            pltpu.sync_copy(q_hbm.at[d_vmem.at[0]], oq_vmem)
            pltpu.sync_copy(kv_hbm.at[s_vmem.at[0]], okv_vmem)

        pltpu.emit_pipeline(
            body,
            grid=(_E_PAD // _WINDOW,),
            in_specs=[pl.BlockSpec((1, _WINDOW), lambda i: (0, i)),
                      pl.BlockSpec((1, _WINDOW), lambda i: (0, i))],
            out_specs=[pl.BlockSpec((_WINDOW, _HALF), lambda i: (i, 0)),
                       pl.BlockSpec((_WINDOW, D_MODEL), lambda i: (i, 0))],
            core_axis_name=("c", "s"),
            dimension_semantics=(pltpu.PARALLEL,),
        )(d_hbm, s_hbm, oq_hbm, okv_hbm)

    return kern(qp, kvp, dst2d, src2d)


def _sc_scatter_add3(valsA, valsB, valsW, idx2d, zeros_sub, c):
    out = jax.ShapeDtypeStruct((_SC_CORES, _ACC_ROWS, _MSG_W), jnp.float32)

    @functools.partial(
        pl.kernel,
        out_type=(out, out, out),
        mesh=_sc_mesh(),
        scratch_types=[pltpu.VMEM_SHARED((_ACC_ROWS, _MSG_W), jnp.float32)])
    def kern(va_hbm, vb_hbm, vw_hbm, i_hbm, z_hbm, oa_hbm, ob_hbm, ow_hbm,
             acc):
        cid = lax.axis_index("c")
        sid = lax.axis_index("s")
        row0 = sid * _ROWS_PER_SUBCORE

        def body(v_vmem, i_vmem):
            pltpu.sync_copy(v_vmem, acc.at[i_vmem.at[0]], add=True)

        ioff = c * (_E_CHUNK // _WINDOW)

        def one_pass(v_hbm, o_hbm):
            pltpu.sync_copy(z_hbm, acc.at[pl.ds(row0, _ROWS_PER_SUBCORE)])
            plsc.subcore_barrier()
            pltpu.emit_pipeline(
                body,
                grid=(_E_CHUNK // _WINDOW,),
                in_specs=[pl.BlockSpec((_WINDOW, _MSG_W), lambda i: (i, 0)),
                          pl.BlockSpec((1, _WINDOW),
                                       lambda i: (0, i + ioff))],
                out_specs=[],
                core_axis_name=("c", "s"),
                dimension_semantics=(pltpu.PARALLEL,),
            )(v_hbm, i_hbm)
            plsc.subcore_barrier()
            pltpu.sync_copy(acc.at[pl.ds(row0, _ROWS_PER_SUBCORE)],
                            o_hbm.at[cid, pl.ds(row0, _ROWS_PER_SUBCORE)])
            plsc.subcore_barrier()

        one_pass(va_hbm, oa_hbm)
        one_pass(vb_hbm, ob_hbm)
        one_pass(vw_hbm, ow_hbm)

    return kern(valsA, valsB, valsW, idx2d, zeros_sub)


def _sc_scatter_add(vals, idx2d, zeros_sub):
    @functools.partial(
        pl.kernel,
        out_type=jax.ShapeDtypeStruct((_SC_CORES, _ACC_ROWS, _MSG_W),
                                      jnp.float32),
        mesh=_sc_mesh(),
        scratch_types=[pltpu.VMEM_SHARED((_ACC_ROWS, _MSG_W), jnp.float32)])
    def kern(v_hbm, i_hbm, z_hbm, o_hbm, acc):
        cid = lax.axis_index("c")
        sid = lax.axis_index("s")
        row0 = sid * _ROWS_PER_SUBCORE
        pltpu.sync_copy(z_hbm, acc.at[pl.ds(row0, _ROWS_PER_SUBCORE)])
        plsc.subcore_barrier()

        def body(v_vmem, i_vmem):
            pltpu.sync_copy(v_vmem, acc.at[i_vmem.at[0]], add=True)

        pltpu.emit_pipeline(
            body,
            grid=(E // _WINDOW,),
            in_specs=[pl.BlockSpec((_WINDOW, _MSG_W), lambda i: (i, 0)),
                      pl.BlockSpec((1, _WINDOW), lambda i: (0, i))],
            out_specs=[],
            core_axis_name=("c", "s"),
            dimension_semantics=(pltpu.PARALLEL,),
        )(v_hbm, i_hbm)

        plsc.subcore_barrier()
        pltpu.sync_copy(acc.at[pl.ds(row0, _ROWS_PER_SUBCORE)],
                        o_hbm.at[cid, pl.ds(row0, _ROWS_PER_SUBCORE)])

    return kern(vals, idx2d, zeros_sub)


def kernel(x, edge_index, edge_attr, atom_table, edge_w, edge_b,
           qkv_w, qkv_b, fc_w, fc_b, ln_w, ln_b):
    pad = jnp.arange(_E_PAD - E, dtype=jnp.int32) % N
    src2d = jnp.concatenate([edge_index[0], pad]).reshape(1, _E_PAD)
    dst2d = jnp.concatenate([edge_index[1], pad]).reshape(1, _E_PAD)
    trash = N + jnp.arange(_E_PAD - E, dtype=jnp.int32) % (_ACC_ROWS - N)
    dst_s = jnp.concatenate([edge_index[1], trash]).reshape(1, _E_PAD)


    perm = jnp.asarray(_P)
    perm_qkv = jnp.concatenate([perm, perm + D_MODEL, perm + 2 * D_MODEL])
    ea = _edge_embed(
        jnp.concatenate([edge_attr, jnp.zeros((_E_PAD - E,),
                                              jnp.float32)]).reshape(_E_PAD, 1),
        edge_w[:, perm], edge_b[perm].reshape(1, D_MODEL))

    table_scaled = atom_table * math.sqrt(D_MODEL)
    x_pad = jnp.concatenate(
        [x, jnp.zeros((_EMB_PAD - N,), jnp.int32)]).reshape(1, _EMB_PAD)
    h = _sc_gather_rows(table_scaled, x_pad)[:N]

    zeros_sub = jnp.zeros((_ROWS_PER_SUBCORE, _MSG_W), jnp.float32)
    for l in range(N_LAYERS):
        qp, kvp = _ln_qkv(h, ln_w[l].reshape(1, -1), ln_b[l].reshape(1, -1),
                          qkv_w[l][:, perm_qkv],
                          qkv_b[l][perm_qkv].reshape(1, -1))
        qdp, kvs = _sc_gather_qkv(qp, kvp, dst2d, src2d)
        parts = []
        for c in range(_NCHUNK):
            msgA, msgB, msgW = _edge_messages(qdp, kvs, ea, c)
            parts.extend(_sc_scatter_add3(msgA, msgB, msgW, dst_s,
                                          zeros_sub, c))
        h = _aggregate_update(h, parts, fc_w[l][perm, :],
                              fc_b[l].reshape(1, -1))
    return h
